# baseline (speedup 1.0000x reference)
import jax
import jax.numpy as jnp
import numpy as np
from jax import lax
from jax.experimental import pallas as pl
from jax.experimental.pallas import tpu as pltpu
from jax.experimental.pallas import tpu_sc as plsc

_E = 8
_ROWS_PER_BLOCK = 256
_NC = 2
_NS = 16
_NW = _NC * _NS


def _f32_gate_kernel(f32_ref):
    r, e, c = f32_ref.shape
    i = pl.program_id(0)
    base = i * r
    f32_ref[...] = jnp.zeros((r, e, c), f32_ref.dtype)
    w = 128
    start = (base // _E) // w * w
    row = base + lax.broadcasted_iota(jnp.int32, (r, e, w), 0)
    exp = lax.broadcasted_iota(jnp.int32, (r, e, w), 1)
    cap = start + lax.broadcasted_iota(jnp.int32, (r, e, w), 2)
    mask = (exp == (row % _E)) & (cap == (row // _E))
    f32_ref[:, :, pl.ds(start, w)] = mask.astype(f32_ref.dtype)


def _sc_mask_kernel(zeros_hbm, diag_hbm, out_hbm, zsem, osem):
    wid = lax.axis_index("s") * _NC + lax.axis_index("c")
    rows = zeros_hbm.shape[0]
    win = diag_hbm.shape[3]
    zcopy = pltpu.async_copy(
        zeros_hbm, out_hbm.at[pl.ds(wid * rows, rows)], zsem)
    zcopy.wait()
    j0 = wid * (rows // _E)
    copies = []
    for t in range(rows // _E):
        j = j0 + t
        l = j % win
        c0 = pl.multiple_of(j - l, win)
        copies.append(pltpu.async_copy(
            diag_hbm.at[l],
            out_hbm.at[pl.ds(_E * j, _E), :, pl.ds(c0, win)],
            osem))
    for c in copies:
        c.wait()


def kernel(input):
    s = input.shape[0]
    capacity = 2 * s // _E
    r = _ROWS_PER_BLOCK
    blk = (r, _E, capacity)
    f32_out = pl.pallas_call(
        _f32_gate_kernel,
        grid=(s // r,),
        out_specs=pl.BlockSpec(blk, lambda i: (i, 0, 0)),
        out_shape=jax.ShapeDtypeStruct((s, _E, capacity), input.dtype),
    )()

    zeros_blk = jnp.zeros((s // _NW, _E, capacity), jnp.uint8)
    dg = np.zeros((128, _E, _E, 128), dtype=np.uint8)
    for l in range(128):
        for r in range(_E):
            dg[l, r, r, l] = 1
    diag = jnp.asarray(dg)
    sc_mask = pl.kernel(
        _sc_mask_kernel,
        out_type=jax.ShapeDtypeStruct((s, _E, capacity), jnp.uint8),
        scratch_types=[
            pltpu.SemaphoreType.DMA,
            pltpu.SemaphoreType.DMA,
        ],
        mesh=plsc.VectorSubcoreMesh(core_axis_name="c", subcore_axis_name="s"),
    )
    mask_u8 = sc_mask(zeros_blk, diag)
    return (0.0, f32_out, mask_u8.astype(jnp.bool_))


# speedup vs baseline: 14.4558x; 14.4558x over previous
import jax
import jax.numpy as jnp
from jax.experimental import pallas as pl

_E = 8
_ROWS_PER_BLOCK = 128


def _rr_gate_kernel(f32_ref, bool_ref):
    r, e, c = f32_ref.shape
    i = pl.program_id(0)
    base = i * r
    f32_ref[...] = jnp.zeros((r, e, c), f32_ref.dtype)
    bool_ref[...] = jnp.zeros((r, e, c), bool_ref.dtype)
    w = 128
    start = (base // _E) // w * w
    row = base + jax.lax.broadcasted_iota(jnp.int32, (r, e, w), 0)
    exp = jax.lax.broadcasted_iota(jnp.int32, (r, e, w), 1)
    cap = start + jax.lax.broadcasted_iota(jnp.int32, (r, e, w), 2)
    mask = (exp == (row % _E)) & (cap == (row // _E))
    f32_ref[:, :, pl.ds(start, w)] = mask.astype(f32_ref.dtype)
    bool_ref[:, :, pl.ds(start, w)] = mask.astype(bool_ref.dtype)


def kernel(input):
    s = input.shape[0]
    capacity = 2 * s // _E
    r = _ROWS_PER_BLOCK
    blk = (r, _E, capacity)
    f32_out, bool_out = pl.pallas_call(
        _rr_gate_kernel,
        grid=(s // r,),
        out_specs=[
            pl.BlockSpec(blk, lambda i: (i, 0, 0)),
            pl.BlockSpec(blk, lambda i: (i, 0, 0)),
        ],
        out_shape=[
            jax.ShapeDtypeStruct((s, _E, capacity), input.dtype),
            jax.ShapeDtypeStruct((s, _E, capacity), jnp.uint8),
        ],
    )()
    return (0.0, f32_out, bool_out.astype(jnp.bool_))
